# Initial kernel scaffold; baseline (speedup 1.0000x reference)
import functools

import jax
import jax.numpy as jnp
from jax import lax
from jax.experimental import pallas as pl
from jax.experimental.pallas import tpu as pltpu
from jax.experimental.pallas import tpu_sc as plsc

NC = 2
NS = 16
NW = NC * NS
C = 80
CNTW = 16


@functools.lru_cache(maxsize=None)
def _sc_aggregate(n_nodes: int, n_edges: int, n_pairs: int, d: int):
    assert n_pairs % (NW * C) == 0
    chunks_per_w = n_pairs // (NW * C)
    rows_per_tile = n_nodes // NS

    mesh = plsc.VectorSubcoreMesh(core_axis_name="c", subcore_axis_name="s")

    @functools.partial(
        pl.kernel,
        mesh=mesh,
        out_type=[
            jax.ShapeDtypeStruct((NC, n_nodes, d), jnp.float32),
            jax.ShapeDtypeStruct((NC, n_nodes, CNTW), jnp.float32),
        ],
        scratch_types=[
            pltpu.VMEM((chunks_per_w, C), jnp.int32),
            pltpu.VMEM((chunks_per_w, C), jnp.int32),
            pltpu.VMEM((C, d), jnp.float32),
            pltpu.VMEM((C, CNTW), jnp.float32),
            pltpu.VMEM_SHARED((n_nodes, d), jnp.float32),
            pltpu.VMEM_SHARED((n_nodes, CNTW), jnp.float32),
            pltpu.SemaphoreType.DMA,
        ],
    )
    def agg(stalks_hbm, eids_hbm, nids_hbm, zacc_hbm, zcnt_hbm, ones_hbm,
            acc_out, cnt_out, idx_e, idx_n, rows, ones, acc_sh, cnt_sh, sem):
        cid = lax.axis_index("c")
        sid = lax.axis_index("s")
        wid = cid * NS + sid

        pltpu.sync_copy(eids_hbm.at[pl.ds(wid * chunks_per_w, chunks_per_w)],
                        idx_e)
        pltpu.sync_copy(nids_hbm.at[pl.ds(wid * chunks_per_w, chunks_per_w)],
                        idx_n)
        pltpu.sync_copy(ones_hbm, ones)

        @pl.when(sid == 0)
        def _init():
            pltpu.sync_copy(zacc_hbm, acc_sh)
            pltpu.sync_copy(zcnt_hbm, cnt_sh)

        plsc.subcore_barrier()

        def body(i, carry):
            pltpu.async_copy(stalks_hbm.at[idx_e.at[i]], rows, sem).wait()
            pltpu.sync_copy(rows, acc_sh.at[idx_n.at[i]], add=True)
            pltpu.sync_copy(ones, cnt_sh.at[idx_n.at[i]], add=True)
            return carry

        lax.fori_loop(0, chunks_per_w, body, 0)

        plsc.subcore_barrier()

        r0 = sid * rows_per_tile
        pltpu.sync_copy(acc_sh.at[pl.ds(r0, rows_per_tile)],
                        acc_out.at[cid, pl.ds(r0, rows_per_tile)])
        pltpu.sync_copy(cnt_sh.at[pl.ds(r0, rows_per_tile)],
                        cnt_out.at[cid, pl.ds(r0, rows_per_tile)])

    return agg


def _tc_body(acc_ref, cnt_ref, nf_ref, w_ref, b_ref, out_ref):
    acc = acc_ref[0] + acc_ref[1]
    c = jnp.sum(cnt_ref[...], axis=1, keepdims=True) * (1.0 / CNTW)
    m = lax.dot_general(acc, w_ref[...], (((1,), (1,)), ((), ())),
                        preferred_element_type=jnp.float32)
    summed = m + c * b_ref[...]
    avg = summed / jnp.maximum(c, 1.0)
    out_ref[...] = jnp.where(c > 0.0, avg, nf_ref[...])


def kernel(node_features, stalks, W, b, edge_ids, node_ids):
    n_nodes, d = node_features.shape
    n_edges = stalks.shape[0]
    n_pairs = edge_ids.shape[0]

    eids2 = edge_ids.reshape(n_pairs // C, C)
    nids2 = node_ids.reshape(n_pairs // C, C)
    zacc = jnp.zeros((n_nodes, d), jnp.float32)
    zcnt = jnp.zeros((n_nodes, CNTW), jnp.float32)
    ones = jnp.ones((C, CNTW), jnp.float32)

    acc_p, cnt_p = _sc_aggregate(n_nodes, n_edges, n_pairs, d)(
        stalks, eids2, nids2, zacc, zcnt, ones)

    cnt_t = jnp.transpose(cnt_p, (1, 0, 2)).reshape(n_nodes, NC * CNTW)

    bn = 1000
    grid = (n_nodes // bn,)
    return pl.pallas_call(
        _tc_body,
        grid=grid,
        in_specs=[
            pl.BlockSpec((NC, bn, d), lambda i: (0, i, 0)),
            pl.BlockSpec((bn, NC * CNTW), lambda i: (i, 0)),
            pl.BlockSpec((bn, d), lambda i: (i, 0)),
            pl.BlockSpec((d, d), lambda i: (0, 0)),
            pl.BlockSpec((1, d), lambda i: (0, 0)),
        ],
        out_specs=pl.BlockSpec((bn, d), lambda i: (i, 0)),
        out_shape=jax.ShapeDtypeStruct((n_nodes, d), jnp.float32),
    )(acc_p, cnt_t, node_features, W, b.reshape(1, d))


# speedup vs baseline: 2.7688x; 2.7688x over previous
import functools

import jax
import jax.numpy as jnp
from jax import lax
from jax.experimental import pallas as pl
from jax.experimental.pallas import tpu as pltpu
from jax.experimental.pallas import tpu_sc as plsc

NC = 2
NS = 16
C = 80


@functools.lru_cache(maxsize=None)
def _sc_aggregate(n_nodes: int, n_pairs: int, d: int):
    assert n_pairs % (NS * C) == 0 and n_nodes % 8 == 0
    chunks_per_t = n_pairs // (NS * C)
    rpt = (n_nodes // NS) & ~7
    tail = n_nodes - NS * rpt

    mesh = plsc.VectorSubcoreMesh(core_axis_name="c", subcore_axis_name="s")

    @functools.partial(
        pl.kernel,
        mesh=mesh,
        out_type=jax.ShapeDtypeStruct((NC, n_nodes, d), jnp.float32),
        scratch_types=[
            pltpu.VMEM((C,), jnp.int32),
            pltpu.VMEM((C,), jnp.int32),
            pltpu.VMEM((C, d), jnp.float32),
            pltpu.VMEM_SHARED((n_nodes, d), jnp.float32),
            pltpu.SemaphoreType.DMA,
        ],
    )
    def agg(stalks_hbm, eids_hbm, nids_hbm, zacc_hbm, ones_hbm,
            out_hbm, idx_e, idx_n, rows, acc_sh, sem):
        cid = lax.axis_index("c")
        sid = lax.axis_index("s")
        p0 = sid * (chunks_per_t * C)

        r0 = sid * rpt
        pltpu.sync_copy(zacc_hbm.at[pl.ds(r0, rpt)],
                        acc_sh.at[pl.ds(r0, rpt)])
        if tail:
            @pl.when(sid == NS - 1)
            def _itail():
                t0 = NS * rpt
                pltpu.sync_copy(zacc_hbm.at[pl.ds(t0, tail)],
                                acc_sh.at[pl.ds(t0, tail)])

        @pl.when(cid == 1)
        def _stage_ones():
            pltpu.sync_copy(ones_hbm, rows)

        plsc.subcore_barrier()

        @pl.when(cid == 0)
        def _features():
            def body(i, carry):
                base = p0 + i * C
                pltpu.sync_copy(eids_hbm.at[pl.ds(base, C)], idx_e)
                pltpu.sync_copy(nids_hbm.at[pl.ds(base, C)], idx_n)
                pltpu.async_copy(stalks_hbm.at[idx_e], rows, sem).wait()
                pltpu.sync_copy(rows, acc_sh.at[idx_n], add=True)
                return carry

            lax.fori_loop(0, chunks_per_t, body, 0)

        @pl.when(cid == 1)
        def _counts():
            def body(i, carry):
                base = p0 + i * C
                pltpu.sync_copy(nids_hbm.at[pl.ds(base, C)], idx_n)
                pltpu.sync_copy(rows, acc_sh.at[idx_n], add=True)
                return carry

            lax.fori_loop(0, chunks_per_t, body, 0)

        plsc.subcore_barrier()

        pltpu.sync_copy(acc_sh.at[pl.ds(r0, rpt)],
                        out_hbm.at[cid, pl.ds(r0, rpt)])
        if tail:
            @pl.when(sid == NS - 1)
            def _otail():
                t0 = NS * rpt
                pltpu.sync_copy(acc_sh.at[pl.ds(t0, tail)],
                                out_hbm.at[cid, pl.ds(t0, tail)])

    return agg


def _tc_body(agg_ref, nf_ref, w_ref, b_ref, out_ref):
    acc = agg_ref[0]
    c = agg_ref[1][:, 0:1]
    m = lax.dot_general(acc, w_ref[...], (((1,), (1,)), ((), ())),
                        preferred_element_type=jnp.float32)
    summed = m + c * b_ref[...]
    avg = summed / jnp.maximum(c, 1.0)
    out_ref[...] = jnp.where(c > 0.0, avg, nf_ref[...])


def kernel(node_features, stalks, W, b, edge_ids, node_ids):
    n_nodes, d = node_features.shape
    n_pairs = edge_ids.shape[0]

    zacc = jnp.zeros((n_nodes, d), jnp.float32)
    ones = jnp.ones((C, d), jnp.float32)

    agg = _sc_aggregate(n_nodes, n_pairs, d)(
        stalks, edge_ids, node_ids, zacc, ones)

    bn = 1000
    grid = (n_nodes // bn,)
    return pl.pallas_call(
        _tc_body,
        grid=grid,
        in_specs=[
            pl.BlockSpec((NC, bn, d), lambda i: (0, i, 0)),
            pl.BlockSpec((bn, d), lambda i: (i, 0)),
            pl.BlockSpec((d, d), lambda i: (0, 0)),
            pl.BlockSpec((1, d), lambda i: (0, 0)),
        ],
        out_specs=pl.BlockSpec((bn, d), lambda i: (i, 0)),
        out_shape=jax.ShapeDtypeStruct((n_nodes, d), jnp.float32),
    )(agg, node_features, W, b.reshape(1, d))


# speedup vs baseline: 4.0300x; 1.4555x over previous
import functools

import jax
import jax.numpy as jnp
from jax import lax
from jax.experimental import pallas as pl
from jax.experimental.pallas import tpu as pltpu
from jax.experimental.pallas import tpu_sc as plsc

NC = 2
NS = 16
NW = NC * NS
C = 80


@functools.lru_cache(maxsize=None)
def _sc_aggregate(n_nodes: int, n_pairs: int, d: int):
    assert n_pairs % (NW * C) == 0 and n_nodes % 8 == 0
    chunks_per_w = n_pairs // (NW * C)
    rpt = (n_nodes // NS) & ~7
    tail = n_nodes - NS * rpt

    mesh = plsc.VectorSubcoreMesh(core_axis_name="c", subcore_axis_name="s")

    @functools.partial(
        pl.kernel,
        mesh=mesh,
        out_type=jax.ShapeDtypeStruct((2 * NC, n_nodes, d), jnp.float32),
        scratch_types=[
            pltpu.VMEM((C,), jnp.int32),
            pltpu.VMEM((C,), jnp.int32),
            pltpu.VMEM((C, d), jnp.float32),
            pltpu.VMEM_SHARED((n_nodes, d), jnp.float32),
            pltpu.SemaphoreType.DMA,
        ],
    )
    def agg(stalks_hbm, eids_hbm, nids_hbm, zacc_hbm, ones_hbm,
            out_hbm, idx_e, idx_n, rows, acc_sh, sem):
        cid = lax.axis_index("c")
        sid = lax.axis_index("s")
        wid = cid * NS + sid
        p0 = wid * (chunks_per_w * C)
        r0 = sid * rpt

        def zero_acc():
            pltpu.sync_copy(zacc_hbm.at[pl.ds(r0, rpt)],
                            acc_sh.at[pl.ds(r0, rpt)])
            if tail:
                @pl.when(sid == NS - 1)
                def _itail():
                    t0 = NS * rpt
                    pltpu.sync_copy(zacc_hbm.at[pl.ds(t0, tail)],
                                    acc_sh.at[pl.ds(t0, tail)])

        def read_out(plane):
            pltpu.sync_copy(acc_sh.at[pl.ds(r0, rpt)],
                            out_hbm.at[plane, pl.ds(r0, rpt)])
            if tail:
                @pl.when(sid == NS - 1)
                def _otail():
                    t0 = NS * rpt
                    pltpu.sync_copy(acc_sh.at[pl.ds(t0, tail)],
                                    out_hbm.at[plane, pl.ds(t0, tail)])

        zero_acc()
        plsc.subcore_barrier()

        def fbody(i, carry):
            base = p0 + i * C
            pltpu.sync_copy(eids_hbm.at[pl.ds(base, C)], idx_e)
            pltpu.sync_copy(nids_hbm.at[pl.ds(base, C)], idx_n)
            pltpu.async_copy(stalks_hbm.at[idx_e], rows, sem).wait()
            pltpu.sync_copy(rows, acc_sh.at[idx_n], add=True)
            return carry

        lax.fori_loop(0, chunks_per_w, fbody, 0)

        plsc.subcore_barrier()
        read_out(cid)

        zero_acc()
        pltpu.sync_copy(ones_hbm, rows)
        plsc.subcore_barrier()

        def cbody(i, carry):
            base = p0 + i * C
            pltpu.sync_copy(nids_hbm.at[pl.ds(base, C)], idx_n)
            pltpu.sync_copy(rows, acc_sh.at[idx_n], add=True)
            return carry

        lax.fori_loop(0, chunks_per_w, cbody, 0)

        plsc.subcore_barrier()
        read_out(NC + cid)

    return agg


def _tc_body(agg_ref, nf_ref, w_ref, b_ref, out_ref):
    acc = agg_ref[0] + agg_ref[1]
    c = (agg_ref[2] + agg_ref[3])[:, 0:1]
    m = lax.dot_general(acc, w_ref[...], (((1,), (1,)), ((), ())),
                        preferred_element_type=jnp.float32)
    summed = m + c * b_ref[...]
    avg = summed / jnp.maximum(c, 1.0)
    out_ref[...] = jnp.where(c > 0.0, avg, nf_ref[...])


def kernel(node_features, stalks, W, b, edge_ids, node_ids):
    n_nodes, d = node_features.shape
    n_pairs = edge_ids.shape[0]

    zacc = jnp.zeros((n_nodes, d), jnp.float32)
    ones = jnp.ones((C, d), jnp.float32)

    agg = _sc_aggregate(n_nodes, n_pairs, d)(
        stalks, edge_ids, node_ids, zacc, ones)

    bn = 1000
    grid = (n_nodes // bn,)
    return pl.pallas_call(
        _tc_body,
        grid=grid,
        in_specs=[
            pl.BlockSpec((2 * NC, bn, d), lambda i: (0, i, 0)),
            pl.BlockSpec((bn, d), lambda i: (i, 0)),
            pl.BlockSpec((d, d), lambda i: (0, 0)),
            pl.BlockSpec((1, d), lambda i: (0, 0)),
        ],
        out_specs=pl.BlockSpec((bn, d), lambda i: (i, 0)),
        out_shape=jax.ShapeDtypeStruct((n_nodes, d), jnp.float32),
    )(agg, node_features, W, b.reshape(1, d))


# speedup vs baseline: 6.9656x; 1.7284x over previous
import functools

import jax
import jax.numpy as jnp
from jax import lax
from jax.experimental import pallas as pl
from jax.experimental.pallas import tpu as pltpu
from jax.experimental.pallas import tpu_sc as plsc

NC = 2
NS = 16
NW = NC * NS
C = 80
K = 8


@functools.lru_cache(maxsize=None)
def _sc_aggregate(n_nodes: int, n_pairs: int, d: int):
    assert n_pairs % (NW * C) == 0 and n_nodes % 8 == 0
    chunks_per_w = n_pairs // (NW * C)
    rpt = (n_nodes // NS) & ~7
    tail = n_nodes - NS * rpt

    mesh = plsc.VectorSubcoreMesh(core_axis_name="c", subcore_axis_name="s")

    @functools.partial(
        pl.kernel,
        mesh=mesh,
        out_type=jax.ShapeDtypeStruct((2 * NC, n_nodes, d), jnp.float32),
        scratch_types=[
            pltpu.VMEM((K, C), jnp.int32),
            pltpu.VMEM((K, C), jnp.int32),
            pltpu.VMEM((2, C, d), jnp.float32),
            pltpu.VMEM((C, d), jnp.float32),
            pltpu.VMEM_SHARED((n_nodes, d), jnp.float32),
            pltpu.SemaphoreType.DMA,
            pltpu.SemaphoreType.DMA,
        ],
    )
    def agg(stalks_hbm, eids_hbm, nids_hbm, zacc_hbm, ones_hbm,
            out_hbm, idx_e, idx_n, rows2, ones_v, acc_sh, sem0, sem1):
        cid = lax.axis_index("c")
        sid = lax.axis_index("s")
        wid = cid * NS + sid
        p0 = wid * (chunks_per_w * C)
        r0 = sid * rpt

        def zero_acc():
            pltpu.sync_copy(zacc_hbm.at[pl.ds(r0, rpt)],
                            acc_sh.at[pl.ds(r0, rpt)])
            if tail:
                @pl.when(sid == NS - 1)
                def _itail():
                    t0 = NS * rpt
                    pltpu.sync_copy(zacc_hbm.at[pl.ds(t0, tail)],
                                    acc_sh.at[pl.ds(t0, tail)])

        def read_out(plane):
            pltpu.sync_copy(acc_sh.at[pl.ds(r0, rpt)],
                            out_hbm.at[plane, pl.ds(r0, rpt)])
            if tail:
                @pl.when(sid == NS - 1)
                def _otail():
                    t0 = NS * rpt
                    pltpu.sync_copy(acc_sh.at[pl.ds(t0, tail)],
                                    out_hbm.at[plane, pl.ds(t0, tail)])

        sems = (sem0, sem1)

        zero_acc()
        plsc.subcore_barrier()

        def fbatch(j, nk):
            pltpu.sync_copy(eids_hbm.at[wid, pl.ds(j * K, nk)],
                            idx_e.at[pl.ds(0, nk)])
            pltpu.sync_copy(nids_hbm.at[wid, pl.ds(j * K, nk)],
                            idx_n.at[pl.ds(0, nk)])
            cps = [None, None]
            cps[0] = pltpu.async_copy(stalks_hbm.at[idx_e.at[0]],
                                      rows2.at[0], sems[0])
            for k in range(nk):
                nxt = k + 1
                if nxt < nk:
                    cps[nxt % 2] = pltpu.async_copy(
                        stalks_hbm.at[idx_e.at[nxt]],
                        rows2.at[nxt % 2], sems[nxt % 2])
                cps[k % 2].wait()
                pltpu.sync_copy(rows2.at[k % 2],
                                acc_sh.at[idx_n.at[k]], add=True)

        nbatch = chunks_per_w // K
        btail = chunks_per_w - nbatch * K

        def fbody(j, carry):
            fbatch(j, K)
            return carry

        lax.fori_loop(0, nbatch, fbody, 0)
        if btail:
            fbatch(nbatch, btail)

        plsc.subcore_barrier()
        read_out(cid)

        zero_acc()
        pltpu.sync_copy(ones_hbm, ones_v)
        plsc.subcore_barrier()

        def cbatch(j, nk):
            pltpu.sync_copy(nids_hbm.at[wid, pl.ds(j * K, nk)],
                            idx_n.at[pl.ds(0, nk)])
            for k in range(nk):
                pltpu.sync_copy(ones_v, acc_sh.at[idx_n.at[k]], add=True)

        def cbody(j, carry):
            cbatch(j, K)
            return carry

        lax.fori_loop(0, nbatch, cbody, 0)
        if btail:
            cbatch(nbatch, btail)

        plsc.subcore_barrier()
        read_out(NC + cid)

    return agg


def _tc_body(agg_ref, nf_ref, w_ref, b_ref, out_ref):
    acc = agg_ref[0] + agg_ref[1]
    c = (agg_ref[2] + agg_ref[3])[:, 0:1]
    m = lax.dot_general(acc, w_ref[...], (((1,), (1,)), ((), ())),
                        preferred_element_type=jnp.float32)
    summed = m + c * b_ref[...]
    avg = summed / jnp.maximum(c, 1.0)
    out_ref[...] = jnp.where(c > 0.0, avg, nf_ref[...])


def kernel(node_features, stalks, W, b, edge_ids, node_ids):
    n_nodes, d = node_features.shape
    n_pairs = edge_ids.shape[0]

    zacc = jnp.zeros((n_nodes, d), jnp.float32)
    ones = jnp.ones((C, d), jnp.float32)
    cw = n_pairs // (NW * C)
    eids3 = edge_ids.reshape(NW, cw, C)
    nids3 = node_ids.reshape(NW, cw, C)

    agg = _sc_aggregate(n_nodes, n_pairs, d)(
        stalks, eids3, nids3, zacc, ones)

    bn = 1000
    grid = (n_nodes // bn,)
    return pl.pallas_call(
        _tc_body,
        grid=grid,
        in_specs=[
            pl.BlockSpec((2 * NC, bn, d), lambda i: (0, i, 0)),
            pl.BlockSpec((bn, d), lambda i: (i, 0)),
            pl.BlockSpec((d, d), lambda i: (0, 0)),
            pl.BlockSpec((1, d), lambda i: (0, 0)),
        ],
        out_specs=pl.BlockSpec((bn, d), lambda i: (i, 0)),
        out_shape=jax.ShapeDtypeStruct((n_nodes, d), jnp.float32),
    )(agg, node_features, W, b.reshape(1, d))


# speedup vs baseline: 7.4069x; 1.0633x over previous
import functools

import jax
import jax.numpy as jnp
from jax import lax
from jax.experimental import pallas as pl
from jax.experimental.pallas import tpu as pltpu
from jax.experimental.pallas import tpu_sc as plsc

NC = 2
NS = 16
NW = NC * NS
C = 80
K = 16


@functools.lru_cache(maxsize=None)
def _sc_aggregate(n_nodes: int, n_pairs: int, d: int):
    assert n_pairs % (NW * C) == 0 and n_nodes % 8 == 0
    chunks_per_w = n_pairs // (NW * C)
    rpt = (n_nodes // NS) & ~7
    tail = n_nodes - NS * rpt

    mesh = plsc.VectorSubcoreMesh(core_axis_name="c", subcore_axis_name="s")

    @functools.partial(
        pl.kernel,
        mesh=mesh,
        out_type=jax.ShapeDtypeStruct((2 * NC, n_nodes, d), jnp.float32),
        scratch_types=[
            pltpu.VMEM((K, C), jnp.int32),
            pltpu.VMEM((K, C), jnp.int32),
            pltpu.VMEM((2, C, d), jnp.float32),
            pltpu.VMEM((C, d), jnp.float32),
            pltpu.VMEM_SHARED((n_nodes, d), jnp.float32),
            pltpu.SemaphoreType.DMA,
            pltpu.SemaphoreType.DMA,
        ],
    )
    def agg(stalks_hbm, eids_hbm, nids_hbm, zacc_hbm, ones_hbm,
            out_hbm, idx_e, idx_n, rows2, ones_v, acc_sh, sem0, sem1):
        cid = lax.axis_index("c")
        sid = lax.axis_index("s")
        wid = cid * NS + sid
        p0 = wid * (chunks_per_w * C)
        r0 = sid * rpt

        def zero_acc():
            pltpu.sync_copy(zacc_hbm.at[pl.ds(r0, rpt)],
                            acc_sh.at[pl.ds(r0, rpt)])
            if tail:
                @pl.when(sid == NS - 1)
                def _itail():
                    t0 = NS * rpt
                    pltpu.sync_copy(zacc_hbm.at[pl.ds(t0, tail)],
                                    acc_sh.at[pl.ds(t0, tail)])

        def read_out(plane):
            pltpu.sync_copy(acc_sh.at[pl.ds(r0, rpt)],
                            out_hbm.at[plane, pl.ds(r0, rpt)])
            if tail:
                @pl.when(sid == NS - 1)
                def _otail():
                    t0 = NS * rpt
                    pltpu.sync_copy(acc_sh.at[pl.ds(t0, tail)],
                                    out_hbm.at[plane, pl.ds(t0, tail)])

        sems = (sem0, sem1)

        zero_acc()
        plsc.subcore_barrier()

        def fbatch(j, nk):
            pltpu.sync_copy(eids_hbm.at[wid, pl.ds(j * K, nk)],
                            idx_e.at[pl.ds(0, nk)])
            pltpu.sync_copy(nids_hbm.at[wid, pl.ds(j * K, nk)],
                            idx_n.at[pl.ds(0, nk)])
            cps = [None, None]
            cps[0] = pltpu.async_copy(stalks_hbm.at[idx_e.at[0]],
                                      rows2.at[0], sems[0])
            for k in range(nk):
                nxt = k + 1
                if nxt < nk:
                    cps[nxt % 2] = pltpu.async_copy(
                        stalks_hbm.at[idx_e.at[nxt]],
                        rows2.at[nxt % 2], sems[nxt % 2])
                cps[k % 2].wait()
                pltpu.sync_copy(rows2.at[k % 2],
                                acc_sh.at[idx_n.at[k]], add=True)

        nbatch = chunks_per_w // K
        btail = chunks_per_w - nbatch * K

        def fbody(j, carry):
            fbatch(j, K)
            return carry

        lax.fori_loop(0, nbatch, fbody, 0)
        if btail:
            fbatch(nbatch, btail)

        plsc.subcore_barrier()
        read_out(cid)

        zero_acc()
        pltpu.sync_copy(ones_hbm, ones_v)
        plsc.subcore_barrier()

        def cbatch(j, nk):
            pltpu.sync_copy(nids_hbm.at[wid, pl.ds(j * K, nk)],
                            idx_n.at[pl.ds(0, nk)])
            cps = [pltpu.async_copy(ones_v, acc_sh.at[idx_n.at[k]], sem0,
                                    add=True)
                   for k in range(nk)]
            for cp in cps:
                cp.wait()

        def cbody(j, carry):
            cbatch(j, K)
            return carry

        lax.fori_loop(0, nbatch, cbody, 0)
        if btail:
            cbatch(nbatch, btail)

        plsc.subcore_barrier()
        read_out(NC + cid)

    return agg


def _tc_body(agg_ref, nf_ref, w_ref, b_ref, out_ref):
    acc = agg_ref[0] + agg_ref[1]
    c = (agg_ref[2] + agg_ref[3])[:, 0:1]
    m = lax.dot_general(acc, w_ref[...], (((1,), (1,)), ((), ())),
                        preferred_element_type=jnp.float32)
    summed = m + c * b_ref[...]
    avg = summed / jnp.maximum(c, 1.0)
    out_ref[...] = jnp.where(c > 0.0, avg, nf_ref[...])


def kernel(node_features, stalks, W, b, edge_ids, node_ids):
    n_nodes, d = node_features.shape
    n_pairs = edge_ids.shape[0]

    zacc = jnp.zeros((n_nodes, d), jnp.float32)
    ones = jnp.ones((C, d), jnp.float32)
    cw = n_pairs // (NW * C)
    eids3 = edge_ids.reshape(NW, cw, C)
    nids3 = node_ids.reshape(NW, cw, C)

    agg = _sc_aggregate(n_nodes, n_pairs, d)(
        stalks, eids3, nids3, zacc, ones)

    bn = 1000
    grid = (n_nodes // bn,)
    return pl.pallas_call(
        _tc_body,
        grid=grid,
        in_specs=[
            pl.BlockSpec((2 * NC, bn, d), lambda i: (0, i, 0)),
            pl.BlockSpec((bn, d), lambda i: (i, 0)),
            pl.BlockSpec((d, d), lambda i: (0, 0)),
            pl.BlockSpec((1, d), lambda i: (0, 0)),
        ],
        out_specs=pl.BlockSpec((bn, d), lambda i: (i, 0)),
        out_shape=jax.ShapeDtypeStruct((n_nodes, d), jnp.float32),
    )(agg, node_features, W, b.reshape(1, d))


# speedup vs baseline: 7.8780x; 1.0636x over previous
import functools

import jax
import jax.numpy as jnp
from jax import lax
from jax.experimental import pallas as pl
from jax.experimental.pallas import tpu as pltpu
from jax.experimental.pallas import tpu_sc as plsc

NC = 2
NS = 16
NW = NC * NS
C = 80
K = 16


@functools.lru_cache(maxsize=None)
def _sc_aggregate(n_nodes: int, n_pairs: int, d: int):
    assert n_pairs % (NW * C) == 0 and n_nodes % 8 == 0
    chunks_per_w = n_pairs // (NW * C)
    rpt = (n_nodes // NS) & ~7
    tail = n_nodes - NS * rpt

    mesh = plsc.VectorSubcoreMesh(core_axis_name="c", subcore_axis_name="s")

    @functools.partial(
        pl.kernel,
        mesh=mesh,
        out_type=jax.ShapeDtypeStruct((2 * NC, n_nodes, d), jnp.float32),
        scratch_types=[
            pltpu.VMEM((K, C), jnp.int32),
            pltpu.VMEM((K, C), jnp.int32),
            pltpu.VMEM((3, C, d), jnp.float32),
            pltpu.VMEM((C, d), jnp.float32),
            pltpu.VMEM_SHARED((n_nodes, d), jnp.float32),
            pltpu.SemaphoreType.DMA,
            pltpu.SemaphoreType.DMA,
            pltpu.SemaphoreType.DMA,
            pltpu.SemaphoreType.DMA,
            pltpu.SemaphoreType.DMA,
            pltpu.SemaphoreType.DMA,
        ],
    )
    def agg(stalks_hbm, eids_hbm, nids_hbm, zacc_hbm, ones_hbm,
            out_hbm, idx_e, idx_n, rows3, ones_v, acc_sh,
            g0, g1, g2, s0, s1, s2):
        cid = lax.axis_index("c")
        sid = lax.axis_index("s")
        wid = cid * NS + sid
        p0 = wid * (chunks_per_w * C)
        r0 = sid * rpt

        def zero_acc():
            pltpu.sync_copy(zacc_hbm.at[pl.ds(r0, rpt)],
                            acc_sh.at[pl.ds(r0, rpt)])
            if tail:
                @pl.when(sid == NS - 1)
                def _itail():
                    t0 = NS * rpt
                    pltpu.sync_copy(zacc_hbm.at[pl.ds(t0, tail)],
                                    acc_sh.at[pl.ds(t0, tail)])

        def read_out(plane):
            pltpu.sync_copy(acc_sh.at[pl.ds(r0, rpt)],
                            out_hbm.at[plane, pl.ds(r0, rpt)])
            if tail:
                @pl.when(sid == NS - 1)
                def _otail():
                    t0 = NS * rpt
                    pltpu.sync_copy(acc_sh.at[pl.ds(t0, tail)],
                                    out_hbm.at[plane, pl.ds(t0, tail)])

        gsems = (g0, g1, g2)
        ssems = (s0, s1, s2)

        zero_acc()
        plsc.subcore_barrier()

        def fbatch(j, nk):
            pltpu.sync_copy(eids_hbm.at[wid, pl.ds(j * K, nk)],
                            idx_e.at[pl.ds(0, nk)])
            pltpu.sync_copy(nids_hbm.at[wid, pl.ds(j * K, nk)],
                            idx_n.at[pl.ds(0, nk)])
            gs = [None, None, None]
            ss = [None, None, None]
            for k in range(min(2, nk)):
                gs[k] = pltpu.async_copy(stalks_hbm.at[idx_e.at[k]],
                                         rows3.at[k], gsems[k])
            for k in range(nk):
                b = k % 3
                gs[b].wait()
                ss[b] = pltpu.async_copy(rows3.at[b],
                                         acc_sh.at[idx_n.at[k]],
                                         ssems[b], add=True)
                nxt = k + 2
                if nxt < nk:
                    nb = nxt % 3
                    if ss[nb] is not None:
                        ss[nb].wait()
                        ss[nb] = None
                    gs[nb] = pltpu.async_copy(stalks_hbm.at[idx_e.at[nxt]],
                                              rows3.at[nb], gsems[nb])
            for b in range(3):
                if ss[b] is not None:
                    ss[b].wait()

        nbatch = chunks_per_w // K
        btail = chunks_per_w - nbatch * K

        def fbody(j, carry):
            fbatch(j, K)
            return carry

        lax.fori_loop(0, nbatch, fbody, 0)
        if btail:
            fbatch(nbatch, btail)

        plsc.subcore_barrier()
        read_out(cid)

        zero_acc()
        pltpu.sync_copy(ones_hbm, ones_v)
        plsc.subcore_barrier()

        def cbatch(j, nk):
            pltpu.sync_copy(nids_hbm.at[wid, pl.ds(j * K, nk)],
                            idx_n.at[pl.ds(0, nk)])
            cps = [pltpu.async_copy(ones_v, acc_sh.at[idx_n.at[k]], s0,
                                    add=True)
                   for k in range(nk)]
            for cp in cps:
                cp.wait()

        def cbody(j, carry):
            cbatch(j, K)
            return carry

        lax.fori_loop(0, nbatch, cbody, 0)
        if btail:
            cbatch(nbatch, btail)

        plsc.subcore_barrier()
        read_out(NC + cid)

    return agg


def _tc_body(agg_ref, nf_ref, w_ref, b_ref, out_ref):
    acc = agg_ref[0] + agg_ref[1]
    c = (agg_ref[2] + agg_ref[3])[:, 0:1]
    m = lax.dot_general(acc, w_ref[...], (((1,), (1,)), ((), ())),
                        preferred_element_type=jnp.float32)
    summed = m + c * b_ref[...]
    avg = summed / jnp.maximum(c, 1.0)
    out_ref[...] = jnp.where(c > 0.0, avg, nf_ref[...])


def kernel(node_features, stalks, W, b, edge_ids, node_ids):
    n_nodes, d = node_features.shape
    n_pairs = edge_ids.shape[0]

    zacc = jnp.zeros((n_nodes, d), jnp.float32)
    ones = jnp.ones((C, d), jnp.float32)
    cw = n_pairs // (NW * C)
    eids3 = edge_ids.reshape(NW, cw, C)
    nids3 = node_ids.reshape(NW, cw, C)

    agg = _sc_aggregate(n_nodes, n_pairs, d)(
        stalks, eids3, nids3, zacc, ones)

    bn = 1000
    grid = (n_nodes // bn,)
    return pl.pallas_call(
        _tc_body,
        grid=grid,
        in_specs=[
            pl.BlockSpec((2 * NC, bn, d), lambda i: (0, i, 0)),
            pl.BlockSpec((bn, d), lambda i: (i, 0)),
            pl.BlockSpec((d, d), lambda i: (0, 0)),
            pl.BlockSpec((1, d), lambda i: (0, 0)),
        ],
        out_specs=pl.BlockSpec((bn, d), lambda i: (i, 0)),
        out_shape=jax.ShapeDtypeStruct((n_nodes, d), jnp.float32),
    )(agg, node_features, W, b.reshape(1, d))
